# baseline (speedup 1.0000x reference)
import functools

import jax
import jax.numpy as jnp
from jax import lax
from jax.experimental import pallas as pl
from jax.experimental.pallas import tpu as pltpu
from jax.experimental.pallas import tpu_sc as plsc

_IN_C = 128
_OUT_C = 128
_K = 15
_SIGMA = 1.0
_H = 32
_DP = 16


_D = _IN_C + _DP


def _sc_gather(table, idx, B, chunk):
    info = plsc.get_sparse_core_info()
    NC, NS = info.num_cores, info.num_subcores
    NW = NC * NS
    b_per_w = B // NW
    n_chunks = b_per_w // chunk
    mesh = plsc.VectorSubcoreMesh(core_axis_name="c", subcore_axis_name="s")

    @functools.partial(
        pl.kernel,
        mesh=mesh,
        out_type=jax.ShapeDtypeStruct((B, _D), jnp.float32),
        scratch_types=[
            pltpu.VMEM((chunk,), jnp.int32),
            pltpu.VMEM((chunk, _D), jnp.float32),
            pltpu.SemaphoreType.DMA,
        ],
        compiler_params=pltpu.CompilerParams(use_tc_tiling_on_sc=False),
    )
    def gather_k(t_hbm, idx_hbm, out_hbm, idx_v, rows_v, sem):
        wid = lax.axis_index("s") * NC + lax.axis_index("c")
        base = wid * b_per_w

        def body(j, carry):
            off = base + j * chunk
            pltpu.sync_copy(idx_hbm.at[pl.ds(off, chunk)], idx_v)
            pltpu.async_copy(t_hbm.at[idx_v], rows_v, sem).wait()
            pltpu.sync_copy(rows_v, out_hbm.at[pl.ds(off, chunk)])
            return carry

        lax.fori_loop(0, n_chunks, body, 0)

    return gather_k(table, idx)


def _tc_body(ft_ref, px_ref, py_ref, pz_ref, qt_ref, kp_ref, wt_ref, o_ref):
    mb = o_ref.shape[2]
    px = px_ref[0] - qt_ref[0, 0, :][None, :]
    py = py_ref[0] - qt_ref[0, 1, :][None, :]
    pz = pz_ref[0] - qt_ref[0, 2, :][None, :]

    def k_step(k, acc):
        dx = px - kp_ref[k, 0]
        dy = py - kp_ref[k, 1]
        dz = pz - kp_ref[k, 2]
        d2 = dx * dx + dy * dy + dz * dz
        infl = jnp.maximum(1.0 - jnp.sqrt(d2) * (1.0 / _SIGMA), 0.0)
        parts = []
        for mt in range(0, mb, 128):
            wf = jnp.zeros((_IN_C, 128), dtype=jnp.float32)
            for h in range(_H):
                wf = wf + (infl[h, mt:mt + 128][None, :]
                           * ft_ref[0, h, :, mt:mt + 128])
            parts.append(wf)
        wf_full = jnp.concatenate(parts, axis=1)
        return acc + jnp.dot(wt_ref[k], wf_full,
                             preferred_element_type=jnp.float32)

    o_ref[0] = lax.fori_loop(
        0, _K, k_step, jnp.zeros((_OUT_C, mb), dtype=jnp.float32))


def _tc_compute(ft, px3, py3, pz3, qt, kp, wt, Mp, Mb):
    n_blocks = Mp // Mb
    coord_spec = pl.BlockSpec((1, _H, Mb), lambda i: (i, 0, 0))
    return pl.pallas_call(
        _tc_body,
        grid=(n_blocks,),
        in_specs=[
            pl.BlockSpec((1, _H, _IN_C, Mb), lambda i: (i, 0, 0, 0)),
            coord_spec,
            coord_spec,
            coord_spec,
            pl.BlockSpec((1, 3, Mb), lambda i: (i, 0, 0)),
            pl.BlockSpec(memory_space=pltpu.SMEM),
            pl.BlockSpec((_K, _OUT_C, _IN_C), lambda i: (0, 0, 0)),
        ],
        out_specs=pl.BlockSpec((1, _OUT_C, Mb), lambda i: (i, 0, 0)),
        out_shape=jax.ShapeDtypeStruct((Mp // Mb, _OUT_C, Mb), jnp.float32),
    )(ft, px3, py3, pz3, qt, kp, wt)


def kernel(q_pts, s_pts, s_feats, neighb_inds, kernel_points, weights):
    M, H = neighb_inds.shape
    N = s_feats.shape[0]
    Mb = 512
    Mp = ((M + Mb - 1) // Mb) * Mb
    nb = Mp // Mb
    B = Mp * H

    table = jnp.concatenate(
        [s_feats, s_pts, jnp.zeros((N, _DP - 3), dtype=jnp.float32)], axis=1)
    inds_p = jnp.pad(neighb_inds, ((0, Mp - M), (0, 0)))
    idx = jnp.transpose(inds_p).reshape(B).astype(jnp.int32)
    g = _sc_gather(table, idx, B, chunk=512)

    g4 = g.reshape(H, nb, Mb, _D)
    ft = jnp.transpose(g4[:, :, :, 0:_IN_C], (1, 0, 3, 2))
    px3 = jnp.transpose(g4[:, :, :, _IN_C + 0], (1, 0, 2))
    py3 = jnp.transpose(g4[:, :, :, _IN_C + 1], (1, 0, 2))
    pz3 = jnp.transpose(g4[:, :, :, _IN_C + 2], (1, 0, 2))
    q_p = jnp.pad(q_pts, ((0, Mp - M), (0, 0)))
    qt = jnp.transpose(q_p.reshape(nb, Mb, 3), (0, 2, 1))
    wt = jnp.transpose(weights[:, 0, :, :], (0, 2, 1))
    out_t = _tc_compute(ft, px3, py3, pz3, qt, kernel_points, wt, Mp, Mb)
    out = jnp.transpose(out_t, (0, 2, 1)).reshape(Mp, _OUT_C)
    return out[:M]


# speedup vs baseline: 1.2176x; 1.2176x over previous
import functools

import jax
import jax.numpy as jnp
from jax import lax
from jax.experimental import pallas as pl
from jax.experimental.pallas import tpu as pltpu
from jax.experimental.pallas import tpu_sc as plsc

_IN_C = 128
_OUT_C = 128
_K = 15
_SIGMA = 1.0
_H = 32
_DP = 16


_D = _IN_C + _DP


def _sc_gather(feats, coords, idx, B, chunk):
    info = plsc.get_sparse_core_info()
    NC, NS = info.num_cores, info.num_subcores
    NW = NC * NS
    b_per_w = B // NW
    n_chunks = b_per_w // chunk
    mesh = plsc.VectorSubcoreMesh(core_axis_name="c", subcore_axis_name="s")

    @functools.partial(
        pl.kernel,
        mesh=mesh,
        out_type=(
            jax.ShapeDtypeStruct((B, _IN_C), jnp.float32),
            jax.ShapeDtypeStruct((B, _DP), jnp.float32),
        ),
        scratch_types=[
            pltpu.VMEM((chunk,), jnp.int32),
            pltpu.VMEM((chunk, _IN_C), jnp.float32),
            pltpu.VMEM((chunk, _DP), jnp.float32),
            pltpu.SemaphoreType.DMA,
            pltpu.SemaphoreType.DMA,
        ],
        compiler_params=pltpu.CompilerParams(use_tc_tiling_on_sc=False),
    )
    def gather_k(f_hbm, c_hbm, idx_hbm, fo_hbm, co_hbm,
                 idx_v, frows_v, crows_v, fsem, csem):
        wid = lax.axis_index("s") * NC + lax.axis_index("c")
        base = wid * b_per_w

        def body(j, carry):
            off = base + j * chunk
            pltpu.sync_copy(idx_hbm.at[pl.ds(off, chunk)], idx_v)
            fcp = pltpu.async_copy(f_hbm.at[idx_v], frows_v, fsem)
            ccp = pltpu.async_copy(c_hbm.at[idx_v], crows_v, csem)
            fcp.wait()
            ccp.wait()
            pltpu.sync_copy(frows_v, fo_hbm.at[pl.ds(off, chunk)])
            pltpu.sync_copy(crows_v, co_hbm.at[pl.ds(off, chunk)])
            return carry

        lax.fori_loop(0, n_chunks, body, 0)

    return gather_k(feats, coords, idx)


def _tc_body(ft_ref, px_ref, py_ref, pz_ref, qt_ref, kp_ref, wt_ref, o_ref):
    mb = o_ref.shape[1]
    px = px_ref[...] - qt_ref[0, :][None, :]
    py = py_ref[...] - qt_ref[1, :][None, :]
    pz = pz_ref[...] - qt_ref[2, :][None, :]

    def k_step(k, acc):
        dx = px - kp_ref[k, 0]
        dy = py - kp_ref[k, 1]
        dz = pz - kp_ref[k, 2]
        d2 = dx * dx + dy * dy + dz * dz
        infl = jnp.maximum(1.0 - jnp.sqrt(d2) * (1.0 / _SIGMA), 0.0)
        parts = []
        for mt in range(0, mb, 128):
            wf = jnp.zeros((_IN_C, 128), dtype=jnp.float32)
            for h in range(_H):
                wf = wf + (infl[h, mt:mt + 128][None, :]
                           * ft_ref[h, :, mt:mt + 128])
            parts.append(wf)
        wf_full = jnp.concatenate(parts, axis=1)
        return acc + jnp.dot(wt_ref[k], wf_full,
                             preferred_element_type=jnp.float32)

    o_ref[...] = lax.fori_loop(
        0, _K, k_step, jnp.zeros((_OUT_C, mb), dtype=jnp.float32))


def _tc_compute(ft, px2, py2, pz2, qt, kp, wt, Mp, Mb):
    n_blocks = Mp // Mb
    coord_spec = pl.BlockSpec((_H, Mb), lambda i: (0, i))
    return pl.pallas_call(
        _tc_body,
        grid=(n_blocks,),
        in_specs=[
            pl.BlockSpec((_H, _IN_C, Mb), lambda i: (0, 0, i)),
            coord_spec,
            coord_spec,
            coord_spec,
            pl.BlockSpec((3, Mb), lambda i: (0, i)),
            pl.BlockSpec(memory_space=pltpu.SMEM),
            pl.BlockSpec((_K, _OUT_C, _IN_C), lambda i: (0, 0, 0)),
        ],
        out_specs=pl.BlockSpec((_OUT_C, Mb), lambda i: (0, i)),
        out_shape=jax.ShapeDtypeStruct((_OUT_C, Mp), jnp.float32),
    )(ft, px2, py2, pz2, qt, kp, wt)


def kernel(q_pts, s_pts, s_feats, neighb_inds, kernel_points, weights):
    M, H = neighb_inds.shape
    N = s_feats.shape[0]
    Mb = 512
    Mp = ((M + Mb - 1) // Mb) * Mb
    nb = Mp // Mb
    B = Mp * H

    coords = jnp.concatenate(
        [s_pts, jnp.zeros((N, _DP - 3), dtype=jnp.float32)], axis=1)
    inds_p = jnp.pad(neighb_inds, ((0, Mp - M), (0, 0)))
    idx = jnp.transpose(inds_p).reshape(B).astype(jnp.int32)
    gf, gc = _sc_gather(s_feats, coords, idx, B, chunk=512)

    gf3 = gf.reshape(H, Mp, _IN_C)
    gc3 = gc.reshape(H, Mp, _DP)
    ft = jnp.transpose(gf3, (0, 2, 1))
    px2 = gc3[:, :, 0]
    py2 = gc3[:, :, 1]
    pz2 = gc3[:, :, 2]
    q_p = jnp.pad(q_pts, ((0, Mp - M), (0, 0)))
    qt = jnp.transpose(q_p)
    wt = jnp.transpose(weights[:, 0, :, :], (0, 2, 1))
    out_t = _tc_compute(ft, px2, py2, pz2, qt, kernel_points, wt, Mp, Mb)
    return jnp.transpose(out_t)[:M]


# speedup vs baseline: 1.2487x; 1.0256x over previous
import functools

import jax
import jax.numpy as jnp
from jax import lax
from jax.experimental import pallas as pl
from jax.experimental.pallas import tpu as pltpu
from jax.experimental.pallas import tpu_sc as plsc

_IN_C = 128
_OUT_C = 128
_K = 15
_SIGMA = 1.0
_H = 32
_DP = 16


_D = _IN_C + _DP


def _sc_gather(feats, coords, idx, B, chunk):
    info = plsc.get_sparse_core_info()
    NC, NS = info.num_cores, info.num_subcores
    NW = NC * NS
    b_per_w = B // NW
    n_chunks = b_per_w // chunk
    mesh = plsc.VectorSubcoreMesh(core_axis_name="c", subcore_axis_name="s")

    @functools.partial(
        pl.kernel,
        mesh=mesh,
        out_type=(
            jax.ShapeDtypeStruct((B, _IN_C), jnp.float32),
            jax.ShapeDtypeStruct((B, _DP), jnp.float32),
        ),
        scratch_types=[
            pltpu.VMEM((2, chunk), jnp.int32),
            pltpu.VMEM((2, chunk, _IN_C), jnp.float32),
            pltpu.VMEM((2, chunk, _DP), jnp.float32),
            pltpu.SemaphoreType.DMA,
            pltpu.SemaphoreType.DMA,
            pltpu.SemaphoreType.DMA,
            pltpu.SemaphoreType.DMA,
        ],
        compiler_params=pltpu.CompilerParams(use_tc_tiling_on_sc=False),
    )
    def gather_k(f_hbm, c_hbm, idx_hbm, fo_hbm, co_hbm,
                 idx_v, frows_v, crows_v, fs0, cs0, fs1, cs1):
        wid = lax.axis_index("s") * NC + lax.axis_index("c")
        base = wid * b_per_w
        fsems = (fs0, fs1)
        csems = (cs0, cs1)

        def start(b, j):
            off = base + j * chunk
            pltpu.sync_copy(idx_hbm.at[pl.ds(off, chunk)], idx_v.at[b])
            pltpu.async_copy(f_hbm.at[idx_v.at[b]], frows_v.at[b], fsems[b])
            pltpu.async_copy(c_hbm.at[idx_v.at[b]], crows_v.at[b], csems[b])

        def finish(b, j):
            off = base + j * chunk
            pltpu.make_async_copy(
                f_hbm.at[idx_v.at[b]], frows_v.at[b], fsems[b]).wait()
            pltpu.make_async_copy(
                c_hbm.at[idx_v.at[b]], crows_v.at[b], csems[b]).wait()
            pltpu.sync_copy(frows_v.at[b], fo_hbm.at[pl.ds(off, chunk)])
            pltpu.sync_copy(crows_v.at[b], co_hbm.at[pl.ds(off, chunk)])

        start(0, 0)
        def body(jj, carry):
            j = jj * 2
            start(1, j + 1)
            finish(0, j)
            start(0, j + 2)
            finish(1, j + 1)
            return carry

        lax.fori_loop(0, n_chunks // 2 - 1, body, 0)
        j_last = n_chunks - 2
        start(1, j_last + 1)
        finish(0, j_last)
        finish(1, j_last + 1)

    return gather_k(feats, coords, idx)


def _tc_body(ft_ref, px_ref, py_ref, pz_ref, qt_ref, kp_ref, wt_ref, o_ref):
    mb = o_ref.shape[1]
    px = px_ref[...] - qt_ref[0, :][None, :]
    py = py_ref[...] - qt_ref[1, :][None, :]
    pz = pz_ref[...] - qt_ref[2, :][None, :]

    def k_step(k, acc):
        dx = px - kp_ref[k, 0]
        dy = py - kp_ref[k, 1]
        dz = pz - kp_ref[k, 2]
        d2 = dx * dx + dy * dy + dz * dz
        infl = jnp.maximum(1.0 - jnp.sqrt(d2) * (1.0 / _SIGMA), 0.0)
        parts = []
        for mt in range(0, mb, 128):
            wf = jnp.zeros((_IN_C, 128), dtype=jnp.float32)
            for h in range(_H):
                wf = wf + (infl[h, mt:mt + 128][None, :]
                           * ft_ref[h, :, mt:mt + 128])
            parts.append(wf)
        wf_full = jnp.concatenate(parts, axis=1)
        return acc + jnp.dot(wt_ref[k], wf_full,
                             preferred_element_type=jnp.float32)

    o_ref[...] = lax.fori_loop(
        0, _K, k_step, jnp.zeros((_OUT_C, mb), dtype=jnp.float32))


def _tc_compute(ft, px2, py2, pz2, qt, kp, wt, Mp, Mb):
    n_blocks = Mp // Mb
    coord_spec = pl.BlockSpec((_H, Mb), lambda i: (0, i))
    return pl.pallas_call(
        _tc_body,
        grid=(n_blocks,),
        in_specs=[
            pl.BlockSpec((_H, _IN_C, Mb), lambda i: (0, 0, i)),
            coord_spec,
            coord_spec,
            coord_spec,
            pl.BlockSpec((3, Mb), lambda i: (0, i)),
            pl.BlockSpec(memory_space=pltpu.SMEM),
            pl.BlockSpec((_K, _OUT_C, _IN_C), lambda i: (0, 0, 0)),
        ],
        out_specs=pl.BlockSpec((_OUT_C, Mb), lambda i: (0, i)),
        out_shape=jax.ShapeDtypeStruct((_OUT_C, Mp), jnp.float32),
    )(ft, px2, py2, pz2, qt, kp, wt)


def kernel(q_pts, s_pts, s_feats, neighb_inds, kernel_points, weights):
    M, H = neighb_inds.shape
    N = s_feats.shape[0]
    Mb = 512
    Mp = ((M + Mb - 1) // Mb) * Mb
    nb = Mp // Mb
    B = Mp * H

    coords = jnp.concatenate(
        [s_pts, jnp.zeros((N, _DP - 3), dtype=jnp.float32)], axis=1)
    inds_p = jnp.pad(neighb_inds, ((0, Mp - M), (0, 0)))
    idx = jnp.transpose(inds_p).reshape(B).astype(jnp.int32)
    gf, gc = _sc_gather(s_feats, coords, idx, B, chunk=256)

    gf3 = gf.reshape(H, Mp, _IN_C)
    gc3 = gc.reshape(H, Mp, _DP)
    ft = jnp.transpose(gf3, (0, 2, 1))
    px2 = gc3[:, :, 0]
    py2 = gc3[:, :, 1]
    pz2 = gc3[:, :, 2]
    q_p = jnp.pad(q_pts, ((0, Mp - M), (0, 0)))
    qt = jnp.transpose(q_p)
    wt = jnp.transpose(weights[:, 0, :, :], (0, 2, 1))
    out_t = _tc_compute(ft, px2, py2, pz2, qt, kernel_points, wt, Mp, Mb)
    return jnp.transpose(out_t)[:M]
